# baseline (speedup 1.0000x reference)
import functools

import jax
import jax.numpy as jnp
from jax import lax
from jax.experimental import pallas as pl
from jax.experimental.pallas import tpu as pltpu
from jax.experimental.pallas import tpu_sc as plsc

_B = 16384
_NC = 2
_NS = 16
_NW = _NC * _NS
_BPW = _B // _NW
_L = 16
_NCH = 2
_CH = _BPW // _NCH

_mesh = plsc.VectorSubcoreMesh(core_axis_name="c", subcore_axis_name="s")


@functools.partial(
    pl.kernel,
    mesh=_mesh,
    out_type=jax.ShapeDtypeStruct((_B,), jnp.float32),
    scratch_types=[
        pltpu.VMEM((_BPW,), jnp.int32),
        pltpu.VMEM((_BPW,), jnp.float32),
        pltpu.VMEM((_BPW,), jnp.float32),
    ]
    + [pltpu.SemaphoreType.DMA] * (2 * _NCH + 2),
)
def _sc_mul_gather(loss_hbm, labels_hbm, weights_hbm, out_hbm,
                   idx_v, w_v, loss_v, *sems):
    lab_sems, g_sems = sems[:_NCH], sems[_NCH:2 * _NCH]
    loss_sem, out_sem = sems[2 * _NCH], sems[2 * _NCH + 1]
    wid = lax.axis_index("s") * _NC + lax.axis_index("c")
    base = wid * _BPW

    lab_cp = [
        pltpu.async_copy(
            labels_hbm.at[pl.ds(base + j * _CH, _CH)],
            idx_v.at[pl.ds(j * _CH, _CH)], lab_sems[j])
        for j in range(_NCH)
    ]
    loss_cp = pltpu.async_copy(loss_hbm.at[pl.ds(base, _BPW)], loss_v, loss_sem)
    g_cp = []
    for j in range(_NCH):
        lab_cp[j].wait()
        g_cp.append(pltpu.async_copy(
            weights_hbm.at[idx_v.at[pl.ds(j * _CH, _CH)]],
            w_v.at[pl.ds(j * _CH, _CH)], g_sems[j]))
    loss_cp.wait()
    out_cp = []
    for j in range(_NCH):
        g_cp[j].wait()
        for i in range(_CH // _L):
            sl = pl.ds(j * _CH + i * _L, _L)
            loss_v[sl] = loss_v[sl] * w_v[sl]
        out_cp.append(pltpu.async_copy(
            loss_v.at[pl.ds(j * _CH, _CH)],
            out_hbm.at[pl.ds(base + j * _CH, _CH)], out_sem))
    for cp in out_cp:
        cp.wait()


def kernel(loss, labels, weights):
    return _sc_mul_gather(loss, labels.astype(jnp.int32), weights)


# speedup vs baseline: 1.0093x; 1.0093x over previous
import functools

import jax
import jax.numpy as jnp
from jax import lax
from jax.experimental import pallas as pl
from jax.experimental.pallas import tpu as pltpu
from jax.experimental.pallas import tpu_sc as plsc

_B = 16384
_NC = 2
_NS = 16
_NW = _NC * _NS
_BPW = _B // _NW
_L = 16
_NCH = 4
_CH = _BPW // _NCH

_mesh = plsc.VectorSubcoreMesh(core_axis_name="c", subcore_axis_name="s")


@functools.partial(
    pl.kernel,
    mesh=_mesh,
    out_type=jax.ShapeDtypeStruct((_B,), jnp.float32),
    scratch_types=[
        pltpu.VMEM((_BPW,), jnp.int32),
        pltpu.VMEM((_BPW,), jnp.float32),
        pltpu.VMEM((_BPW,), jnp.float32),
    ]
    + [pltpu.SemaphoreType.DMA] * (2 * _NCH + 2),
)
def _sc_mul_gather(loss_hbm, labels_hbm, weights_hbm, out_hbm,
                   idx_v, w_v, loss_v, *sems):
    lab_sems, g_sems = sems[:_NCH], sems[_NCH:2 * _NCH]
    loss_sem, out_sem = sems[2 * _NCH], sems[2 * _NCH + 1]
    wid = lax.axis_index("s") * _NC + lax.axis_index("c")
    base = wid * _BPW

    lab_cp = [
        pltpu.async_copy(
            labels_hbm.at[pl.ds(base + j * _CH, _CH)],
            idx_v.at[pl.ds(j * _CH, _CH)], lab_sems[j])
        for j in range(_NCH)
    ]
    loss_cp = pltpu.async_copy(loss_hbm.at[pl.ds(base, _BPW)], loss_v, loss_sem)
    g_cp = []
    for j in range(_NCH):
        lab_cp[j].wait()
        g_cp.append(pltpu.async_copy(
            weights_hbm.at[idx_v.at[pl.ds(j * _CH, _CH)]],
            w_v.at[pl.ds(j * _CH, _CH)], g_sems[j]))
    loss_cp.wait()
    out_cp = []
    for j in range(_NCH):
        g_cp[j].wait()
        for i in range(_CH // _L):
            sl = pl.ds(j * _CH + i * _L, _L)
            loss_v[sl] = loss_v[sl] * w_v[sl]
        out_cp.append(pltpu.async_copy(
            loss_v.at[pl.ds(j * _CH, _CH)],
            out_hbm.at[pl.ds(base + j * _CH, _CH)], out_sem))
    for cp in out_cp:
        cp.wait()


def kernel(loss, labels, weights):
    return _sc_mul_gather(loss, labels.astype(jnp.int32), weights)
